# Initial kernel scaffold; baseline (speedup 1.0000x reference)
import jax
import jax.numpy as jnp
from jax.experimental import pallas as pl


def kernel(node_emb, basis, comp, root_W, root_b, attn_Wa, attn_a, fc1_W, fc1_b, bn_gamma, bn_beta, fc2_W, fc2_b, edge_index, edge_type, user_ids):
    raise NotImplementedError("write your pallas kernel here")


# speedup vs baseline: 2.0866x; 2.0866x over previous
import functools

import jax
import jax.numpy as jnp
from jax import lax
from jax.experimental import pallas as pl
from jax.experimental.pallas import tpu as pltpu
from jax.experimental.pallas import tpu_sc as plsc

N_ENTITY = 10000
N_EDGES = 160000
N_REL = 48
NUM_BASES = 8
D = 128
BATCH = 1024
HIST = 50

NC, NS = 2, 16
NW = NC * NS
N_PAD = 10240
E_PAD = 163840
E_ROWS_W = E_PAD // NW // 128
AGG_ROWS_S = N_PAD // NS
U_PAD = 53248
U_ROWS_W = U_PAD // NW // 128

NODE_BLK = 256
DEG_W = 16


def _wcat_body(basist_ref, comp_ref, w_ref):
    r = pl.program_id(0)
    acc = comp_ref[r, 0] * basist_ref[:, 0, :]
    for b in range(1, NUM_BASES):
        acc = acc + comp_ref[r, b] * basist_ref[:, b, :]
    w_ref[...] = acc


def _wcat(basis_t, comp):
    return pl.pallas_call(
        _wcat_body,
        grid=(N_REL,),
        in_specs=[
            pl.BlockSpec((D, NUM_BASES, D), lambda r: (0, 0, 0)),
            pl.BlockSpec(memory_space=pltpu.SMEM),
        ],
        out_specs=pl.BlockSpec((D, D), lambda r: (0, r)),
        out_shape=jax.ShapeDtypeStruct((D, N_REL * D), jnp.float32),
    )(basis_t, comp)


def _project_body(x_ref, wcat_ref, rootw_ref, rootb_ref, z_ref, kgroot_ref):
    x = x_ref[...]
    z_ref[...] = jnp.dot(x, wcat_ref[...],
                         preferred_element_type=jnp.float32)
    kgroot_ref[...] = (
        jnp.dot(x, rootw_ref[...], preferred_element_type=jnp.float32)
        + rootb_ref[...])


def _project(node_emb, wcat, root_w, root_b):
    grid = N_PAD // NODE_BLK
    return pl.pallas_call(
        _project_body,
        grid=(grid,),
        in_specs=[
            pl.BlockSpec((NODE_BLK, D), lambda i: (i, 0)),
            pl.BlockSpec((D, N_REL * D), lambda i: (0, 0)),
            pl.BlockSpec((D, D), lambda i: (0, 0)),
            pl.BlockSpec((1, D), lambda i: (0, 0)),
        ],
        out_specs=[
            pl.BlockSpec((NODE_BLK, N_REL * D), lambda i: (i, 0)),
            pl.BlockSpec((NODE_BLK, D), lambda i: (i, 0)),
        ],
        out_shape=[
            jax.ShapeDtypeStruct((N_PAD, N_REL * D), jnp.float32),
            jax.ShapeDtypeStruct((N_PAD, D), jnp.float32),
        ],
    )(node_emb, wcat, root_w, root_b)


def _edge_scatter_body(src_hbm, et_hbm, dst_hbm, z_hbm,
                       agg_hbm, deg_hbm,
                       src_v, et_v, dst_v, gidx_v, rows_v, ones_v, zrow_v,
                       agg_sh, deg_sh, sem):
    c = lax.axis_index("c")
    s = lax.axis_index("s")
    wid = c * NS + s

    def _zrows(i, _):
        def _inner(j, _):
            rows_v[i, pl.ds(j * 16, 16)] = jnp.zeros((16,), jnp.float32)
            return 0
        return lax.fori_loop(0, D // 16, _inner, 0)
    lax.fori_loop(0, 128, _zrows, 0)

    def _zrow(j, _):
        zrow_v[pl.ds(j * 16, 16)] = jnp.zeros((16,), jnp.float32)
        return 0
    lax.fori_loop(0, AGG_ROWS_S // 16, _zrow, 0)

    def _ones(j, _):
        ones_v[pl.ds(j * 16, 16)] = jnp.ones((16,), jnp.float32)
        return 0
    lax.fori_loop(0, 128 // 16, _ones, 0)

    def _zinit(k, _):
        pltpu.sync_copy(rows_v, agg_sh.at[pl.ds(s * AGG_ROWS_S + k * 128, 128)])
        return 0
    lax.fori_loop(0, AGG_ROWS_S // 128, _zinit, 0)
    pltpu.sync_copy(zrow_v, deg_sh.at[pl.ds(s * AGG_ROWS_S, AGG_ROWS_S)])
    plsc.subcore_barrier()

    rowbase = wid * E_ROWS_W
    pltpu.sync_copy(src_hbm.at[pl.ds(rowbase, E_ROWS_W)], src_v)
    pltpu.sync_copy(et_hbm.at[pl.ds(rowbase, E_ROWS_W)], et_v)
    pltpu.sync_copy(dst_hbm.at[pl.ds(rowbase, E_ROWS_W)], dst_v)

    def _gidx_row(i, _):
        def _inner(j, _):
            sl = pl.ds(j * 16, 16)
            gidx_v[i, sl] = src_v[i, sl] * N_REL + et_v[i, sl]
            return 0
        return lax.fori_loop(0, D // 16, _inner, 0)
    lax.fori_loop(0, E_ROWS_W, _gidx_row, 0)

    def _edge_chunk(i, _):
        pltpu.async_copy(z_hbm.at[gidx_v.at[i]], rows_v, sem).wait()
        pltpu.sync_copy(rows_v, agg_sh.at[dst_v.at[i]], add=True)
        pltpu.sync_copy(ones_v, deg_sh.at[dst_v.at[i]], add=True)
        return 0
    lax.fori_loop(0, E_ROWS_W, _edge_chunk, 0)
    plsc.subcore_barrier()

    def _out(k, _):
        sl = pl.ds(s * AGG_ROWS_S + k * 128, 128)
        pltpu.sync_copy(agg_sh.at[sl], agg_hbm.at[c, sl])
        return 0
    lax.fori_loop(0, AGG_ROWS_S // 128, _out, 0)
    sl = pl.ds(s * AGG_ROWS_S, AGG_ROWS_S)
    pltpu.sync_copy(deg_sh.at[sl], deg_hbm.at[c, sl])


@functools.cache
def _sc_mesh():
    return plsc.VectorSubcoreMesh(core_axis_name="c", subcore_axis_name="s",
                                  num_cores=NC, num_subcores=NS)


@functools.cache
def _edge_scatter_fn():
    return pl.kernel(
        _edge_scatter_body,
        out_type=[
            jax.ShapeDtypeStruct((NC, N_PAD, D), jnp.float32),
            jax.ShapeDtypeStruct((NC, N_PAD), jnp.float32),
        ],
        mesh=_sc_mesh(),
        scratch_types=[
        pltpu.VMEM((E_ROWS_W, 128), jnp.int32),
        pltpu.VMEM((E_ROWS_W, 128), jnp.int32),
        pltpu.VMEM((E_ROWS_W, 128), jnp.int32),
        pltpu.VMEM((E_ROWS_W, 128), jnp.int32),
        pltpu.VMEM((128, D), jnp.float32),
        pltpu.VMEM((128,), jnp.float32),
        pltpu.VMEM((AGG_ROWS_S,), jnp.float32),
            pltpu.VMEM_SHARED((N_PAD, D), jnp.float32),
            pltpu.VMEM_SHARED((N_PAD,), jnp.float32),
            pltpu.SemaphoreType.DMA,
        ],
    )


def _combine_body(agg_ref, deg_ref, kgroot_ref, kg_ref):
    a = agg_ref[0] + agg_ref[1]
    dg = jnp.maximum(deg_ref[:, 0:1] + deg_ref[:, 1:2], 1.0)
    kg_ref[...] = a / dg + kgroot_ref[...]


def _combine(agg, deg, kg_root):
    blk = 1024
    return pl.pallas_call(
        _combine_body,
        grid=(N_PAD // blk,),
        in_specs=[
            pl.BlockSpec((NC, blk, D), lambda i: (0, i, 0)),
            pl.BlockSpec((blk, NC), lambda i: (i, 0)),
            pl.BlockSpec((blk, D), lambda i: (i, 0)),
        ],
        out_specs=pl.BlockSpec((blk, D), lambda i: (i, 0)),
        out_shape=jax.ShapeDtypeStruct((N_PAD, D), jnp.float32),
    )(agg, deg, kg_root)


def _user_gather_body(uidx_hbm, kg_hbm, h_hbm, uidx_v, rows_v, sem):
    c = lax.axis_index("c")
    s = lax.axis_index("s")
    wid = c * NS + s
    base = wid * U_ROWS_W * 128
    pltpu.sync_copy(uidx_hbm.at[pl.ds(base, U_ROWS_W * 128)], uidx_v)

    def _chunk(i, _):
        pltpu.async_copy(kg_hbm.at[uidx_v.at[pl.ds(i * 128, 128)]],
                         rows_v, sem).wait()
        pltpu.sync_copy(rows_v, h_hbm.at[pl.ds(base + i * 128, 128)])
        return 0
    lax.fori_loop(0, U_ROWS_W, _chunk, 0)


@functools.cache
def _user_gather_fn():
    return pl.kernel(
        _user_gather_body,
        out_type=jax.ShapeDtypeStruct((U_PAD, D), jnp.float32),
        mesh=_sc_mesh(),
        scratch_types=[
            pltpu.VMEM((U_ROWS_W * 128,), jnp.int32),
            pltpu.VMEM((128, D), jnp.float32),
            pltpu.SemaphoreType.DMA,
        ],
    )


B_BLK = 128


def _attn_body(h_ref, wa_ref, a_ref, fc1w_ref, fc1b_ref, out_ref):
    flat = h_ref[...]
    t = jnp.tanh(jnp.dot(flat, wa_ref[...],
                         preferred_element_type=jnp.float32))
    e = jax.lax.dot_general(t, a_ref[...], (((1,), (1,)), ((), ())),
                            preferred_element_type=jnp.float32)
    e2 = e.reshape(B_BLK, HIST)
    e2 = e2 - jnp.max(e2, axis=1, keepdims=True)
    ex = jnp.exp(e2)
    alpha = ex / jnp.sum(ex, axis=1, keepdims=True)
    hb = flat.reshape(B_BLK, HIST, D)
    prof = jnp.zeros((B_BLK, D), jnp.float32)
    for l in range(HIST):
        prof = prof + alpha[:, l][:, None] * hb[:, l, :]
    out_ref[...] = (jnp.dot(prof, fc1w_ref[...],
                            preferred_element_type=jnp.float32)
                    + fc1b_ref[...])


def _attn_fc1(h_gathered, attn_wa, attn_a, fc1_w, fc1_b):
    return pl.pallas_call(
        _attn_body,
        grid=(BATCH // B_BLK,),
        in_specs=[
            pl.BlockSpec((B_BLK * HIST, D), lambda i: (i, 0)),
            pl.BlockSpec((D, D), lambda i: (0, 0)),
            pl.BlockSpec((1, D), lambda i: (0, 0)),
            pl.BlockSpec((D, D), lambda i: (0, 0)),
            pl.BlockSpec((1, D), lambda i: (0, 0)),
        ],
        out_specs=pl.BlockSpec((B_BLK, D), lambda i: (i, 0)),
        out_shape=jax.ShapeDtypeStruct((BATCH, D), jnp.float32),
    )(h_gathered, attn_wa, attn_a, fc1_w, fc1_b)


def _head_body(h_ref, gamma_ref, beta_ref, fc2w_ref, fc2b_ref, out_ref):
    h = h_ref[...]
    mu = jnp.mean(h, axis=0, keepdims=True)
    var = jnp.mean((h - mu) * (h - mu), axis=0, keepdims=True)
    hn = (h - mu) * lax.rsqrt(var + 1e-5) * gamma_ref[...] + beta_ref[...]
    hr = jnp.maximum(hn, 0.0)
    out_ref[...] = (jnp.dot(hr, fc2w_ref[...],
                            preferred_element_type=jnp.float32)
                    + fc2b_ref[...])


def _bn_fc2(h, gamma, beta, fc2_w, fc2_b):
    return pl.pallas_call(
        _head_body,
        out_shape=jax.ShapeDtypeStruct((BATCH, D), jnp.float32),
    )(h, gamma, beta, fc2_w, fc2_b)


def kernel(node_emb, basis, comp, root_W, root_b, attn_Wa, attn_a,
           fc1_W, fc1_b, bn_gamma, bn_beta, fc2_W, fc2_b,
           edge_index, edge_type, user_ids):
    basis_t = jnp.transpose(basis, (1, 0, 2))
    wcat = _wcat(basis_t, comp)
    node_emb_p = jnp.pad(node_emb, ((0, N_PAD - N_ENTITY), (0, 0)))
    z, kg_root = _project(node_emb_p, wcat, root_W, root_b.reshape(1, D))
    z_flat = z.reshape(N_PAD * N_REL, D)

    src = edge_index[0].astype(jnp.int32)
    dst = edge_index[1].astype(jnp.int32)
    et = edge_type.astype(jnp.int32)
    npad = E_PAD - N_EDGES
    pad_iota = jnp.arange(npad, dtype=jnp.int32)
    src_p = jnp.concatenate([src, pad_iota % N_ENTITY]).reshape(-1, 128)
    et_p = jnp.concatenate([et, jnp.zeros((npad,), jnp.int32)]).reshape(-1, 128)
    dst_p = jnp.concatenate(
        [dst, N_ENTITY + pad_iota % (N_PAD - N_ENTITY)]).reshape(-1, 128)

    agg, deg = _edge_scatter_fn()(src_p, et_p, dst_p, z_flat)
    kg = _combine(agg, jnp.transpose(deg), kg_root)

    uflat = user_ids.reshape(-1).astype(jnp.int32)
    upad = jnp.concatenate(
        [uflat,
         jnp.arange(U_PAD - BATCH * HIST, dtype=jnp.int32) % N_ENTITY])
    h_gathered = _user_gather_fn()(upad, kg)

    h = _attn_fc1(h_gathered, attn_Wa, attn_a.reshape(1, D),
                  fc1_W, fc1_b.reshape(1, D))
    return _bn_fc2(h, bn_gamma.reshape(1, D), bn_beta.reshape(1, D),
                   fc2_W, fc2_b.reshape(1, D))


# speedup vs baseline: 10.3453x; 4.9579x over previous
import functools

import jax
import jax.numpy as jnp
from jax import lax
from jax.experimental import pallas as pl
from jax.experimental.pallas import tpu as pltpu
from jax.experimental.pallas import tpu_sc as plsc

N_ENTITY = 10000
N_EDGES = 160000
N_REL = 48
NUM_BASES = 8
D = 128
BATCH = 1024
HIST = 50

NC, NS = 2, 16
NW = NC * NS
N_PAD = 10240
E_PAD = 163840
E_ROWS_W = E_PAD // NW // 128
AGG_ROWS_S = N_PAD // NS
U_PAD = 53248
U_ROWS_W = U_PAD // NW // 128

NODE_BLK = 256
DEG_W = 16


def _wcat_body(basist_ref, comp_ref, w_ref):
    r = pl.program_id(0)
    acc = comp_ref[r, 0] * basist_ref[:, 0, :]
    for b in range(1, NUM_BASES):
        acc = acc + comp_ref[r, b] * basist_ref[:, b, :]
    w_ref[...] = acc


def _wcat(basis_t, comp):
    return pl.pallas_call(
        _wcat_body,
        grid=(N_REL,),
        in_specs=[
            pl.BlockSpec((D, NUM_BASES, D), lambda r: (0, 0, 0)),
            pl.BlockSpec(memory_space=pltpu.SMEM),
        ],
        out_specs=pl.BlockSpec((D, D), lambda r: (0, r)),
        out_shape=jax.ShapeDtypeStruct((D, N_REL * D), jnp.float32),
    )(basis_t, comp)


def _project_body(x_ref, wcat_ref, z_ref):
    z_ref[...] = jnp.dot(x_ref[...], wcat_ref[...],
                         preferred_element_type=jnp.float32)


def _project(node_emb, wcat):
    return pl.pallas_call(
        _project_body,
        grid=(N_REL,),
        in_specs=[
            pl.BlockSpec((N_PAD, D), lambda r: (0, 0)),
            pl.BlockSpec((D, D), lambda r: (0, r)),
        ],
        out_specs=pl.BlockSpec((N_PAD, D), lambda r: (r, 0)),
        out_shape=jax.ShapeDtypeStruct((N_REL * N_PAD, D), jnp.float32),
    )(node_emb, wcat)


def _edge_scatter_body(src_hbm, et_hbm, dst_hbm, z_hbm,
                       agg_hbm, deg_hbm,
                       src_v, et_v, dst_v, gidx_v, rows_v, ones_v, zrow_v,
                       agg_sh, deg_sh, sem):
    c = lax.axis_index("c")
    s = lax.axis_index("s")
    wid = c * NS + s

    def _zrows(i, _):
        def _inner(j, _):
            rows_v[i, pl.ds(j * 16, 16)] = jnp.zeros((16,), jnp.float32)
            return 0
        return lax.fori_loop(0, D // 16, _inner, 0)
    lax.fori_loop(0, 128, _zrows, 0)

    def _zrow(j, _):
        zrow_v[pl.ds(j * 16, 16)] = jnp.zeros((16,), jnp.float32)
        return 0
    lax.fori_loop(0, AGG_ROWS_S // 16, _zrow, 0)

    def _ones(j, _):
        ones_v[pl.ds(j * 16, 16)] = jnp.ones((16,), jnp.float32)
        return 0
    lax.fori_loop(0, 128 // 16, _ones, 0)

    def _zinit(k, _):
        pltpu.sync_copy(rows_v, agg_sh.at[pl.ds(s * AGG_ROWS_S + k * 128, 128)])
        return 0
    lax.fori_loop(0, AGG_ROWS_S // 128, _zinit, 0)
    pltpu.sync_copy(zrow_v, deg_sh.at[pl.ds(s * AGG_ROWS_S, AGG_ROWS_S)])
    plsc.subcore_barrier()

    rowbase = wid * E_ROWS_W
    pltpu.sync_copy(src_hbm.at[pl.ds(rowbase, E_ROWS_W)], src_v)
    pltpu.sync_copy(et_hbm.at[pl.ds(rowbase, E_ROWS_W)], et_v)
    pltpu.sync_copy(dst_hbm.at[pl.ds(rowbase, E_ROWS_W)], dst_v)

    def _gidx_row(i, _):
        def _inner(j, _):
            sl = pl.ds(j * 16, 16)
            gidx_v[i, sl] = et_v[i, sl] * N_PAD + src_v[i, sl]
            return 0
        return lax.fori_loop(0, D // 16, _inner, 0)
    lax.fori_loop(0, E_ROWS_W, _gidx_row, 0)

    def _edge_chunk(i, _):
        pltpu.async_copy(z_hbm.at[gidx_v.at[i]], rows_v, sem).wait()
        pltpu.sync_copy(rows_v, agg_sh.at[dst_v.at[i]], add=True)
        pltpu.sync_copy(ones_v, deg_sh.at[dst_v.at[i]], add=True)
        return 0
    lax.fori_loop(0, E_ROWS_W, _edge_chunk, 0)
    plsc.subcore_barrier()

    def _out(k, _):
        sl = pl.ds(s * AGG_ROWS_S + k * 128, 128)
        pltpu.sync_copy(agg_sh.at[sl], agg_hbm.at[c, sl])
        return 0
    lax.fori_loop(0, AGG_ROWS_S // 128, _out, 0)
    sl = pl.ds(s * AGG_ROWS_S, AGG_ROWS_S)
    pltpu.sync_copy(deg_sh.at[sl], deg_hbm.at[c, sl])


@functools.cache
def _sc_mesh():
    return plsc.VectorSubcoreMesh(core_axis_name="c", subcore_axis_name="s",
                                  num_cores=NC, num_subcores=NS)


@functools.cache
def _edge_scatter_fn():
    return pl.kernel(
        _edge_scatter_body,
        out_type=[
            jax.ShapeDtypeStruct((NC, N_PAD, D), jnp.float32),
            jax.ShapeDtypeStruct((NC, N_PAD), jnp.float32),
        ],
        mesh=_sc_mesh(),
        scratch_types=[
        pltpu.VMEM((E_ROWS_W, 128), jnp.int32),
        pltpu.VMEM((E_ROWS_W, 128), jnp.int32),
        pltpu.VMEM((E_ROWS_W, 128), jnp.int32),
        pltpu.VMEM((E_ROWS_W, 128), jnp.int32),
        pltpu.VMEM((128, D), jnp.float32),
        pltpu.VMEM((128,), jnp.float32),
        pltpu.VMEM((AGG_ROWS_S,), jnp.float32),
            pltpu.VMEM_SHARED((N_PAD, D), jnp.float32),
            pltpu.VMEM_SHARED((N_PAD,), jnp.float32),
            pltpu.SemaphoreType.DMA,
        ],
    )


def _combine_body(agg_ref, deg_ref, x_ref, rootw_ref, rootb_ref, kg_ref):
    a = agg_ref[0] + agg_ref[1]
    dg = jnp.maximum(deg_ref[:, 0:1] + deg_ref[:, 1:2], 1.0)
    root = (jnp.dot(x_ref[...], rootw_ref[...],
                    preferred_element_type=jnp.float32) + rootb_ref[...])
    kg_ref[...] = a / dg + root


def _combine(agg, deg, node_emb, root_w, root_b):
    blk = 1024
    return pl.pallas_call(
        _combine_body,
        grid=(N_PAD // blk,),
        in_specs=[
            pl.BlockSpec((NC, blk, D), lambda i: (0, i, 0)),
            pl.BlockSpec((blk, NC), lambda i: (i, 0)),
            pl.BlockSpec((blk, D), lambda i: (i, 0)),
            pl.BlockSpec((D, D), lambda i: (0, 0)),
            pl.BlockSpec((1, D), lambda i: (0, 0)),
        ],
        out_specs=pl.BlockSpec((blk, D), lambda i: (i, 0)),
        out_shape=jax.ShapeDtypeStruct((N_PAD, D), jnp.float32),
    )(agg, deg, node_emb, root_w, root_b)


def _user_gather_body(uidx_hbm, kg_hbm, h_hbm, uidx_v, rows_v, sem):
    c = lax.axis_index("c")
    s = lax.axis_index("s")
    wid = c * NS + s
    base = wid * U_ROWS_W * 128
    pltpu.sync_copy(uidx_hbm.at[pl.ds(base, U_ROWS_W * 128)], uidx_v)

    def _chunk(i, _):
        pltpu.async_copy(kg_hbm.at[uidx_v.at[pl.ds(i * 128, 128)]],
                         rows_v, sem).wait()
        pltpu.sync_copy(rows_v, h_hbm.at[pl.ds(base + i * 128, 128)])
        return 0
    lax.fori_loop(0, U_ROWS_W, _chunk, 0)


@functools.cache
def _user_gather_fn():
    return pl.kernel(
        _user_gather_body,
        out_type=jax.ShapeDtypeStruct((U_PAD, D), jnp.float32),
        mesh=_sc_mesh(),
        scratch_types=[
            pltpu.VMEM((U_ROWS_W * 128,), jnp.int32),
            pltpu.VMEM((128, D), jnp.float32),
            pltpu.SemaphoreType.DMA,
        ],
    )


B_BLK = 128


def _attn_body(h_ref, wa_ref, a_ref, mask_ref, fc1w_ref, fc1b_ref, out_ref):
    flat = h_ref[...]
    t = jnp.tanh(jnp.dot(flat, wa_ref[...],
                         preferred_element_type=jnp.float32))
    e = jnp.dot(t, a_ref[...], preferred_element_type=jnp.float32)
    ex = jnp.exp(e)
    m = mask_ref[...]
    s = jnp.dot(m, ex, preferred_element_type=jnp.float32)
    praw = jnp.dot(m, ex * flat,
                   preferred_element_type=jnp.float32)
    prof = praw / s
    out_ref[...] = (jnp.dot(prof, fc1w_ref[...],
                            preferred_element_type=jnp.float32)
                    + fc1b_ref[...])


def _attn_fc1(h_gathered, attn_wa, attn_a, mask, fc1_w, fc1_b):
    return pl.pallas_call(
        _attn_body,
        grid=(BATCH // B_BLK,),
        in_specs=[
            pl.BlockSpec((B_BLK * HIST, D), lambda i: (i, 0)),
            pl.BlockSpec((D, D), lambda i: (0, 0)),
            pl.BlockSpec((D, 1), lambda i: (0, 0)),
            pl.BlockSpec((B_BLK, B_BLK * HIST), lambda i: (0, 0)),
            pl.BlockSpec((D, D), lambda i: (0, 0)),
            pl.BlockSpec((1, D), lambda i: (0, 0)),
        ],
        out_specs=pl.BlockSpec((B_BLK, D), lambda i: (i, 0)),
        out_shape=jax.ShapeDtypeStruct((BATCH, D), jnp.float32),
    )(h_gathered, attn_wa, attn_a, mask, fc1_w, fc1_b)


def _head_body(h_ref, gamma_ref, beta_ref, fc2w_ref, fc2b_ref, out_ref):
    h = h_ref[...]
    mu = jnp.mean(h, axis=0, keepdims=True)
    var = jnp.mean((h - mu) * (h - mu), axis=0, keepdims=True)
    hn = (h - mu) * lax.rsqrt(var + 1e-5) * gamma_ref[...] + beta_ref[...]
    hr = jnp.maximum(hn, 0.0)
    out_ref[...] = (jnp.dot(hr, fc2w_ref[...],
                            preferred_element_type=jnp.float32)
                    + fc2b_ref[...])


def _bn_fc2(h, gamma, beta, fc2_w, fc2_b):
    return pl.pallas_call(
        _head_body,
        out_shape=jax.ShapeDtypeStruct((BATCH, D), jnp.float32),
    )(h, gamma, beta, fc2_w, fc2_b)


def kernel(node_emb, basis, comp, root_W, root_b, attn_Wa, attn_a,
           fc1_W, fc1_b, bn_gamma, bn_beta, fc2_W, fc2_b,
           edge_index, edge_type, user_ids):
    basis_t = jnp.transpose(basis, (1, 0, 2))
    wcat = _wcat(basis_t, comp)
    node_emb_p = jnp.pad(node_emb, ((0, N_PAD - N_ENTITY), (0, 0)))
    z_flat = _project(node_emb_p, wcat)

    src = edge_index[0].astype(jnp.int32)
    dst = edge_index[1].astype(jnp.int32)
    et = edge_type.astype(jnp.int32)
    npad = E_PAD - N_EDGES
    pad_iota = jnp.arange(npad, dtype=jnp.int32)
    src_p = jnp.concatenate([src, pad_iota % N_ENTITY]).reshape(-1, 128)
    et_p = jnp.concatenate([et, jnp.zeros((npad,), jnp.int32)]).reshape(-1, 128)
    dst_p = jnp.concatenate(
        [dst, N_ENTITY + pad_iota % (N_PAD - N_ENTITY)]).reshape(-1, 128)

    agg, deg = _edge_scatter_fn()(src_p, et_p, dst_p, z_flat)
    kg = _combine(agg, jnp.transpose(deg), node_emb_p, root_W,
                  root_b.reshape(1, D))

    uflat = user_ids.reshape(-1).astype(jnp.int32)
    upad = jnp.concatenate(
        [uflat,
         jnp.arange(U_PAD - BATCH * HIST, dtype=jnp.int32) % N_ENTITY])
    h_gathered = _user_gather_fn()(upad, kg)

    mask = jnp.repeat(jnp.eye(B_BLK, dtype=jnp.float32), HIST, axis=1)
    h = _attn_fc1(h_gathered, attn_Wa, attn_a.reshape(D, 1), mask,
                  fc1_W, fc1_b.reshape(1, D))
    return _bn_fc2(h, bn_gamma.reshape(1, D), bn_beta.reshape(1, D),
                   fc2_W, fc2_b.reshape(1, D))


# speedup vs baseline: 12.1907x; 1.1784x over previous
import functools

import jax
import jax.numpy as jnp
from jax import lax
from jax.experimental import pallas as pl
from jax.experimental.pallas import tpu as pltpu
from jax.experimental.pallas import tpu_sc as plsc

N_ENTITY = 10000
N_EDGES = 160000
N_REL = 48
NUM_BASES = 8
D = 128
BATCH = 1024
HIST = 50

NC, NS = 2, 16
NW = NC * NS
N_PAD = 10240
E_PAD = 163840
E_ROWS_W = E_PAD // NW // 128
AGG_ROWS_S = N_PAD // NS
U_PAD = 53248
U_ROWS_W = U_PAD // NW // 128

NODE_BLK = 256
DEG_W = 16


def _project_body(x_ref, basist_ref, comp_ref, z_ref):
    r = pl.program_id(0)
    w = comp_ref[r, 0] * basist_ref[:, 0, :]
    for b in range(1, NUM_BASES):
        w = w + comp_ref[r, b] * basist_ref[:, b, :]
    z_ref[...] = jnp.dot(x_ref[...], w,
                         preferred_element_type=jnp.float32)


def _project(node_emb, basis_t, comp):
    return pl.pallas_call(
        _project_body,
        grid=(N_REL,),
        in_specs=[
            pl.BlockSpec((N_PAD, D), lambda r: (0, 0)),
            pl.BlockSpec((D, NUM_BASES, D), lambda r: (0, 0, 0)),
            pl.BlockSpec(memory_space=pltpu.SMEM),
        ],
        out_specs=pl.BlockSpec((N_PAD, D), lambda r: (r, 0)),
        out_shape=jax.ShapeDtypeStruct((N_REL * N_PAD, D), jnp.float32),
    )(node_emb, basis_t, comp)


def _edge_scatter_body(gidx_hbm, dst_hbm, z_hbm,
                       agg_hbm, deg_hbm,
                       dst_v, gidx_v, rows_v, rows2_v, ones_v,
                       zrow_v, agg_sh, deg_sh, sem):
    c = lax.axis_index("c")
    s = lax.axis_index("s")
    wid = c * NS + s

    def _zrows(i, _):
        def _inner(j, _):
            rows_v[i, pl.ds(j * 16, 16)] = jnp.zeros((16,), jnp.float32)
            return 0
        return lax.fori_loop(0, D // 16, _inner, 0)
    lax.fori_loop(0, 128, _zrows, 0)

    def _zrow(j, _):
        zrow_v[pl.ds(j * 16, 16)] = jnp.zeros((16,), jnp.float32)
        return 0
    lax.fori_loop(0, AGG_ROWS_S // 16, _zrow, 0)

    def _ones(j, _):
        ones_v[pl.ds(j * 16, 16)] = jnp.ones((16,), jnp.float32)
        return 0
    lax.fori_loop(0, 128 // 16, _ones, 0)

    def _zinit(k, _):
        pltpu.sync_copy(rows_v, agg_sh.at[pl.ds(s * AGG_ROWS_S + k * 128, 128)])
        return 0
    lax.fori_loop(0, AGG_ROWS_S // 128, _zinit, 0)
    pltpu.sync_copy(zrow_v, deg_sh.at[pl.ds(s * AGG_ROWS_S, AGG_ROWS_S)])
    plsc.subcore_barrier()

    rowbase = wid * E_ROWS_W
    pltpu.sync_copy(gidx_hbm.at[pl.ds(rowbase, E_ROWS_W)], gidx_v)
    pltpu.sync_copy(dst_hbm.at[pl.ds(rowbase, E_ROWS_W)], dst_v)

    pltpu.async_copy(z_hbm.at[gidx_v.at[0]], rows_v, sem)

    def _edge_pair(i2, _):
        i = i2 * 2
        pltpu.make_async_copy(z_hbm.at[gidx_v.at[i]], rows_v, sem).wait()
        pltpu.async_copy(z_hbm.at[gidx_v.at[i + 1]], rows2_v, sem)
        pltpu.sync_copy(rows_v, agg_sh.at[dst_v.at[i]], add=True)
        pltpu.sync_copy(ones_v, deg_sh.at[dst_v.at[i]], add=True)
        pltpu.make_async_copy(z_hbm.at[gidx_v.at[i + 1]], rows2_v, sem).wait()

        @pl.when(i2 < E_ROWS_W // 2 - 1)
        def _():
            pltpu.async_copy(z_hbm.at[gidx_v.at[i + 2]], rows_v, sem)
        pltpu.sync_copy(rows2_v, agg_sh.at[dst_v.at[i + 1]], add=True)
        pltpu.sync_copy(ones_v, deg_sh.at[dst_v.at[i + 1]], add=True)
        return 0
    lax.fori_loop(0, E_ROWS_W // 2, _edge_pair, 0)
    plsc.subcore_barrier()

    def _out(k, _):
        sl = pl.ds(s * AGG_ROWS_S + k * 128, 128)
        pltpu.sync_copy(agg_sh.at[sl], agg_hbm.at[c, sl])
        return 0
    lax.fori_loop(0, AGG_ROWS_S // 128, _out, 0)
    sl = pl.ds(s * AGG_ROWS_S, AGG_ROWS_S)
    pltpu.sync_copy(deg_sh.at[sl], deg_hbm.at[c, sl])


@functools.cache
def _sc_mesh():
    return plsc.VectorSubcoreMesh(core_axis_name="c", subcore_axis_name="s",
                                  num_cores=NC, num_subcores=NS)


@functools.cache
def _edge_scatter_fn():
    return pl.kernel(
        _edge_scatter_body,
        out_type=[
            jax.ShapeDtypeStruct((NC, N_PAD, D), jnp.float32),
            jax.ShapeDtypeStruct((NC, N_PAD), jnp.float32),
        ],
        mesh=_sc_mesh(),
        scratch_types=[
        pltpu.VMEM((E_ROWS_W, 128), jnp.int32),
        pltpu.VMEM((E_ROWS_W, 128), jnp.int32),
        pltpu.VMEM((128, D), jnp.float32),
        pltpu.VMEM((128, D), jnp.float32),
        pltpu.VMEM((128,), jnp.float32),
        pltpu.VMEM((AGG_ROWS_S,), jnp.float32),
            pltpu.VMEM_SHARED((N_PAD, D), jnp.float32),
            pltpu.VMEM_SHARED((N_PAD,), jnp.float32),
            pltpu.SemaphoreType.DMA,
        ],
    )


def _combine_body(agg_ref, deg_ref, x_ref, rootw_ref, rootb_ref, kg_ref):
    a = agg_ref[0] + agg_ref[1]
    dg = jnp.maximum(deg_ref[:, 0:1] + deg_ref[:, 1:2], 1.0)
    root = (jnp.dot(x_ref[...], rootw_ref[...],
                    preferred_element_type=jnp.float32) + rootb_ref[...])
    kg_ref[...] = a / dg + root


def _combine(agg, deg, node_emb, root_w, root_b):
    blk = 1024
    return pl.pallas_call(
        _combine_body,
        grid=(N_PAD // blk,),
        in_specs=[
            pl.BlockSpec((NC, blk, D), lambda i: (0, i, 0)),
            pl.BlockSpec((blk, NC), lambda i: (i, 0)),
            pl.BlockSpec((blk, D), lambda i: (i, 0)),
            pl.BlockSpec((D, D), lambda i: (0, 0)),
            pl.BlockSpec((1, D), lambda i: (0, 0)),
        ],
        out_specs=pl.BlockSpec((blk, D), lambda i: (i, 0)),
        out_shape=jax.ShapeDtypeStruct((N_PAD, D), jnp.float32),
    )(agg, deg, node_emb, root_w, root_b)


def _user_gather_body(uidx_hbm, kg_hbm, h_hbm, uidx_v, rows_v, sem):
    c = lax.axis_index("c")
    s = lax.axis_index("s")
    wid = c * NS + s
    base = wid * U_ROWS_W * 128
    pltpu.sync_copy(uidx_hbm.at[pl.ds(base, U_ROWS_W * 128)], uidx_v)

    def _chunk(i, _):
        pltpu.async_copy(kg_hbm.at[uidx_v.at[pl.ds(i * 128, 128)]],
                         rows_v, sem).wait()
        pltpu.sync_copy(rows_v, h_hbm.at[pl.ds(base + i * 128, 128)])
        return 0
    lax.fori_loop(0, U_ROWS_W, _chunk, 0)


@functools.cache
def _user_gather_fn():
    return pl.kernel(
        _user_gather_body,
        out_type=jax.ShapeDtypeStruct((U_PAD, D), jnp.float32),
        mesh=_sc_mesh(),
        scratch_types=[
            pltpu.VMEM((U_ROWS_W * 128,), jnp.int32),
            pltpu.VMEM((128, D), jnp.float32),
            pltpu.SemaphoreType.DMA,
        ],
    )


B_BLK = 128


def _attn_body(h_ref, wa_ref, a_ref, mask_ref, fc1w_ref, fc1b_ref, out_ref):
    flat = h_ref[...]
    t = jnp.tanh(jnp.dot(flat, wa_ref[...],
                         preferred_element_type=jnp.float32))
    e = jnp.dot(t, a_ref[...], preferred_element_type=jnp.float32)
    ex = jnp.exp(e)
    m = mask_ref[...]
    s = jnp.dot(m, ex, preferred_element_type=jnp.float32)
    praw = jnp.dot(m, ex * flat,
                   preferred_element_type=jnp.float32)
    prof = praw / s
    out_ref[...] = (jnp.dot(prof, fc1w_ref[...],
                            preferred_element_type=jnp.float32)
                    + fc1b_ref[...])


def _attn_fc1(h_gathered, attn_wa, attn_a, mask, fc1_w, fc1_b):
    return pl.pallas_call(
        _attn_body,
        grid=(BATCH // B_BLK,),
        in_specs=[
            pl.BlockSpec((B_BLK * HIST, D), lambda i: (i, 0)),
            pl.BlockSpec((D, D), lambda i: (0, 0)),
            pl.BlockSpec((D, 1), lambda i: (0, 0)),
            pl.BlockSpec((B_BLK, B_BLK * HIST), lambda i: (0, 0)),
            pl.BlockSpec((D, D), lambda i: (0, 0)),
            pl.BlockSpec((1, D), lambda i: (0, 0)),
        ],
        out_specs=pl.BlockSpec((B_BLK, D), lambda i: (i, 0)),
        out_shape=jax.ShapeDtypeStruct((BATCH, D), jnp.float32),
    )(h_gathered, attn_wa, attn_a, mask, fc1_w, fc1_b)


def _head_body(h_ref, gamma_ref, beta_ref, fc2w_ref, fc2b_ref, out_ref):
    h = h_ref[...]
    mu = jnp.mean(h, axis=0, keepdims=True)
    var = jnp.mean((h - mu) * (h - mu), axis=0, keepdims=True)
    hn = (h - mu) * lax.rsqrt(var + 1e-5) * gamma_ref[...] + beta_ref[...]
    hr = jnp.maximum(hn, 0.0)
    out_ref[...] = (jnp.dot(hr, fc2w_ref[...],
                            preferred_element_type=jnp.float32)
                    + fc2b_ref[...])


def _bn_fc2(h, gamma, beta, fc2_w, fc2_b):
    return pl.pallas_call(
        _head_body,
        out_shape=jax.ShapeDtypeStruct((BATCH, D), jnp.float32),
    )(h, gamma, beta, fc2_w, fc2_b)


def kernel(node_emb, basis, comp, root_W, root_b, attn_Wa, attn_a,
           fc1_W, fc1_b, bn_gamma, bn_beta, fc2_W, fc2_b,
           edge_index, edge_type, user_ids):
    basis_t = jnp.transpose(basis, (1, 0, 2))
    node_emb_p = jnp.pad(node_emb, ((0, N_PAD - N_ENTITY), (0, 0)))
    z_flat = _project(node_emb_p, basis_t, comp)

    src = edge_index[0].astype(jnp.int32)
    dst = edge_index[1].astype(jnp.int32)
    et = edge_type.astype(jnp.int32)
    npad = E_PAD - N_EDGES
    pad_iota = jnp.arange(npad, dtype=jnp.int32)
    gidx_p = jnp.concatenate(
        [et * N_PAD + src, pad_iota % N_ENTITY]).reshape(-1, 128)
    dst_p = jnp.concatenate(
        [dst, N_ENTITY + pad_iota % (N_PAD - N_ENTITY)]).reshape(-1, 128)

    agg, deg = _edge_scatter_fn()(gidx_p, dst_p, z_flat)
    kg = _combine(agg, jnp.transpose(deg), node_emb_p, root_W,
                  root_b.reshape(1, D))

    uflat = user_ids.reshape(-1).astype(jnp.int32)
    upad = jnp.concatenate(
        [uflat,
         jnp.arange(U_PAD - BATCH * HIST, dtype=jnp.int32) % N_ENTITY])
    h_gathered = _user_gather_fn()(upad, kg)

    mask = jnp.repeat(jnp.eye(B_BLK, dtype=jnp.float32), HIST, axis=1)
    h = _attn_fc1(h_gathered, attn_Wa, attn_a.reshape(D, 1), mask,
                  fc1_W, fc1_b.reshape(1, D))
    return _bn_fc2(h, bn_gamma.reshape(1, D), bn_beta.reshape(1, D),
                   fc2_W, fc2_b.reshape(1, D))


# speedup vs baseline: 12.4216x; 1.0189x over previous
import functools

import numpy as np

import jax
import jax.numpy as jnp
from jax import lax
from jax.experimental import pallas as pl
from jax.experimental.pallas import tpu as pltpu
from jax.experimental.pallas import tpu_sc as plsc

N_ENTITY = 10000
N_EDGES = 160000
N_REL = 48
NUM_BASES = 8
D = 128
BATCH = 1024
HIST = 50

NC, NS = 2, 16
NW = NC * NS
N_PAD = 10240
E_PAD = 163840
E_ROWS_W = E_PAD // NW // 128
AGG_ROWS_S = N_PAD // NS
U_PAD = 53248
U_ROWS_W = U_PAD // NW // 128

NODE_BLK = 256
DEG_W = 16


def _project_body(x_ref, basist_ref, comp_ref, z_ref):
    r = pl.program_id(0)
    w = comp_ref[r, 0] * basist_ref[:, 0, :]
    for b in range(1, NUM_BASES):
        w = w + comp_ref[r, b] * basist_ref[:, b, :]
    z_ref[...] = jnp.dot(x_ref[...], w,
                         preferred_element_type=jnp.float32)


def _project(node_emb, basis_t, comp):
    return pl.pallas_call(
        _project_body,
        grid=(N_REL,),
        in_specs=[
            pl.BlockSpec((N_PAD, D), lambda r: (0, 0)),
            pl.BlockSpec((D, NUM_BASES, D), lambda r: (0, 0, 0)),
            pl.BlockSpec(memory_space=pltpu.SMEM),
        ],
        out_specs=pl.BlockSpec((N_PAD, D), lambda r: (r, 0)),
        out_shape=jax.ShapeDtypeStruct((N_REL * N_PAD, D), jnp.float32),
    )(node_emb, basis_t, comp)


def _edge_scatter_body(gidx_hbm, dst_hbm, z_hbm,
                       agg_hbm, deg_hbm,
                       dst_v, gidx_v, rows_v, rows2_v, ones_v,
                       zrow_v, agg_sh, deg_sh, sem, sem2):
    c = lax.axis_index("c")
    s = lax.axis_index("s")
    wid = c * NS + s

    def _zrows(i, _):
        def _inner(j, _):
            rows_v[i, pl.ds(j * 16, 16)] = jnp.zeros((16,), jnp.float32)
            return 0
        return lax.fori_loop(0, D // 16, _inner, 0)
    lax.fori_loop(0, 128, _zrows, 0)

    def _zrow(j, _):
        zrow_v[pl.ds(j * 16, 16)] = jnp.zeros((16,), jnp.float32)
        return 0
    lax.fori_loop(0, AGG_ROWS_S // 16, _zrow, 0)

    def _ones(j, _):
        ones_v[pl.ds(j * 16, 16)] = jnp.ones((16,), jnp.float32)
        return 0
    lax.fori_loop(0, 128 // 16, _ones, 0)

    def _zinit(k, _):
        pltpu.sync_copy(rows_v, agg_sh.at[pl.ds(s * AGG_ROWS_S + k * 128, 128)])
        return 0
    lax.fori_loop(0, AGG_ROWS_S // 128, _zinit, 0)
    pltpu.sync_copy(zrow_v, deg_sh.at[pl.ds(s * AGG_ROWS_S, AGG_ROWS_S)])
    plsc.subcore_barrier()

    rowbase = wid * E_ROWS_W
    pltpu.sync_copy(gidx_hbm.at[pl.ds(rowbase, E_ROWS_W)], gidx_v)
    pltpu.sync_copy(dst_hbm.at[pl.ds(rowbase, E_ROWS_W)], dst_v)

    pltpu.async_copy(z_hbm.at[gidx_v.at[0]], rows_v, sem)

    def _edge_pair(i2, _):
        i = i2 * 2
        pltpu.make_async_copy(z_hbm.at[gidx_v.at[i]], rows_v, sem).wait()
        pltpu.async_copy(z_hbm.at[gidx_v.at[i + 1]], rows2_v, sem)
        pltpu.sync_copy(rows_v, agg_sh.at[dst_v.at[i]], add=True)
        pltpu.async_copy(ones_v, deg_sh.at[dst_v.at[i]], sem2, add=True)
        pltpu.make_async_copy(z_hbm.at[gidx_v.at[i + 1]], rows2_v, sem).wait()

        @pl.when(i2 < E_ROWS_W // 2 - 1)
        def _():
            pltpu.async_copy(z_hbm.at[gidx_v.at[i + 2]], rows_v, sem)
        pltpu.sync_copy(rows2_v, agg_sh.at[dst_v.at[i + 1]], add=True)
        pltpu.async_copy(ones_v, deg_sh.at[dst_v.at[i + 1]], sem2, add=True)
        return 0
    lax.fori_loop(0, E_ROWS_W // 2, _edge_pair, 0)

    def _deg_drain(i, _):
        pltpu.make_async_copy(ones_v, deg_sh.at[dst_v.at[0]], sem2).wait()
        return 0
    lax.fori_loop(0, E_ROWS_W, _deg_drain, 0)
    plsc.subcore_barrier()

    def _out(k, _):
        sl = pl.ds(s * AGG_ROWS_S + k * 128, 128)
        pltpu.sync_copy(agg_sh.at[sl], agg_hbm.at[c, sl])
        return 0
    lax.fori_loop(0, AGG_ROWS_S // 128, _out, 0)
    sl = pl.ds(s * AGG_ROWS_S, AGG_ROWS_S)
    pltpu.sync_copy(deg_sh.at[sl], deg_hbm.at[c, sl])


@functools.cache
def _sc_mesh():
    return plsc.VectorSubcoreMesh(core_axis_name="c", subcore_axis_name="s",
                                  num_cores=NC, num_subcores=NS)


@functools.cache
def _edge_scatter_fn():
    return pl.kernel(
        _edge_scatter_body,
        out_type=[
            jax.ShapeDtypeStruct((NC, N_PAD, D), jnp.float32),
            jax.ShapeDtypeStruct((NC, N_PAD), jnp.float32),
        ],
        mesh=_sc_mesh(),
        scratch_types=[
        pltpu.VMEM((E_ROWS_W, 128), jnp.int32),
        pltpu.VMEM((E_ROWS_W, 128), jnp.int32),
        pltpu.VMEM((128, D), jnp.float32),
        pltpu.VMEM((128, D), jnp.float32),
        pltpu.VMEM((128,), jnp.float32),
        pltpu.VMEM((AGG_ROWS_S,), jnp.float32),
            pltpu.VMEM_SHARED((N_PAD, D), jnp.float32),
            pltpu.VMEM_SHARED((N_PAD,), jnp.float32),
            pltpu.SemaphoreType.DMA,
            pltpu.SemaphoreType.DMA,
        ],
    )


def _combine_body(agg_ref, deg_ref, x_ref, rootw_ref, rootb_ref, kg_ref):
    a = agg_ref[0] + agg_ref[1]
    dg = jnp.maximum(deg_ref[:, 0:1] + deg_ref[:, 1:2], 1.0)
    root = (jnp.dot(x_ref[...], rootw_ref[...],
                    preferred_element_type=jnp.float32) + rootb_ref[...])
    kg_ref[...] = a / dg + root


def _combine(agg, deg, node_emb, root_w, root_b):
    blk = 1024
    return pl.pallas_call(
        _combine_body,
        grid=(N_PAD // blk,),
        in_specs=[
            pl.BlockSpec((NC, blk, D), lambda i: (0, i, 0)),
            pl.BlockSpec((blk, NC), lambda i: (i, 0)),
            pl.BlockSpec((blk, D), lambda i: (i, 0)),
            pl.BlockSpec((D, D), lambda i: (0, 0)),
            pl.BlockSpec((1, D), lambda i: (0, 0)),
        ],
        out_specs=pl.BlockSpec((blk, D), lambda i: (i, 0)),
        out_shape=jax.ShapeDtypeStruct((N_PAD, D), jnp.float32),
    )(agg, deg, node_emb, root_w, root_b)


def _user_gather_body(uidx_hbm, kg_hbm, h_hbm, uidx_v, rows_v, rows2_v, sem):
    c = lax.axis_index("c")
    s = lax.axis_index("s")
    wid = c * NS + s
    base = wid * U_ROWS_W * 128
    pltpu.sync_copy(uidx_hbm.at[pl.ds(base, U_ROWS_W * 128)], uidx_v)

    bufs = (rows_v, rows2_v)
    pltpu.async_copy(kg_hbm.at[uidx_v.at[pl.ds(0, 128)]], bufs[0], sem)
    for i in range(U_ROWS_W):
        b = bufs[i % 2]
        pltpu.make_async_copy(kg_hbm.at[uidx_v.at[pl.ds(i * 128, 128)]],
                              b, sem).wait()
        if i + 1 < U_ROWS_W:
            pltpu.async_copy(
                kg_hbm.at[uidx_v.at[pl.ds((i + 1) * 128, 128)]],
                bufs[(i + 1) % 2], sem)
        pltpu.sync_copy(b, h_hbm.at[pl.ds(base + i * 128, 128)])


@functools.cache
def _user_gather_fn():
    return pl.kernel(
        _user_gather_body,
        out_type=jax.ShapeDtypeStruct((U_PAD, D), jnp.float32),
        mesh=_sc_mesh(),
        scratch_types=[
            pltpu.VMEM((U_ROWS_W * 128,), jnp.int32),
            pltpu.VMEM((128, D), jnp.float32),
            pltpu.VMEM((128, D), jnp.float32),
            pltpu.SemaphoreType.DMA,
        ],
    )


B_BLK = 128
_POOL_MASK = np.repeat(np.eye(B_BLK, dtype=np.float32), HIST, axis=1)


def _attn_body(h_ref, wa_ref, a_ref, mask_ref, fc1w_ref, fc1b_ref, out_ref):
    flat = h_ref[...]
    t = jnp.tanh(jnp.dot(flat, wa_ref[...],
                         preferred_element_type=jnp.float32))
    e = jnp.dot(t, a_ref[...], preferred_element_type=jnp.float32)
    ex = jnp.exp(e)
    m = mask_ref[...]
    s = jnp.dot(m, ex, preferred_element_type=jnp.float32)
    praw = jnp.dot(m, ex * flat,
                   preferred_element_type=jnp.float32)
    prof = praw / s
    out_ref[...] = (jnp.dot(prof, fc1w_ref[...],
                            preferred_element_type=jnp.float32)
                    + fc1b_ref[...])


def _attn_fc1(h_gathered, attn_wa, attn_a, mask, fc1_w, fc1_b):
    return pl.pallas_call(
        _attn_body,
        grid=(BATCH // B_BLK,),
        in_specs=[
            pl.BlockSpec((B_BLK * HIST, D), lambda i: (i, 0)),
            pl.BlockSpec((D, D), lambda i: (0, 0)),
            pl.BlockSpec((D, 1), lambda i: (0, 0)),
            pl.BlockSpec((B_BLK, B_BLK * HIST), lambda i: (0, 0)),
            pl.BlockSpec((D, D), lambda i: (0, 0)),
            pl.BlockSpec((1, D), lambda i: (0, 0)),
        ],
        out_specs=pl.BlockSpec((B_BLK, D), lambda i: (i, 0)),
        out_shape=jax.ShapeDtypeStruct((BATCH, D), jnp.float32),
    )(h_gathered, attn_wa, attn_a, mask, fc1_w, fc1_b)


def _head_body(h_ref, gamma_ref, beta_ref, fc2w_ref, fc2b_ref, out_ref):
    h = h_ref[...]
    mu = jnp.mean(h, axis=0, keepdims=True)
    var = jnp.mean((h - mu) * (h - mu), axis=0, keepdims=True)
    hn = (h - mu) * lax.rsqrt(var + 1e-5) * gamma_ref[...] + beta_ref[...]
    hr = jnp.maximum(hn, 0.0)
    out_ref[...] = (jnp.dot(hr, fc2w_ref[...],
                            preferred_element_type=jnp.float32)
                    + fc2b_ref[...])


def _bn_fc2(h, gamma, beta, fc2_w, fc2_b):
    return pl.pallas_call(
        _head_body,
        out_shape=jax.ShapeDtypeStruct((BATCH, D), jnp.float32),
    )(h, gamma, beta, fc2_w, fc2_b)


def kernel(node_emb, basis, comp, root_W, root_b, attn_Wa, attn_a,
           fc1_W, fc1_b, bn_gamma, bn_beta, fc2_W, fc2_b,
           edge_index, edge_type, user_ids):
    basis_t = jnp.transpose(basis, (1, 0, 2))
    node_emb_p = jnp.pad(node_emb, ((0, N_PAD - N_ENTITY), (0, 0)))
    z_flat = _project(node_emb_p, basis_t, comp)

    src = edge_index[0].astype(jnp.int32)
    dst = edge_index[1].astype(jnp.int32)
    et = edge_type.astype(jnp.int32)
    npad = E_PAD - N_EDGES
    pad_iota = jnp.arange(npad, dtype=jnp.int32)
    gidx_p = jnp.concatenate(
        [et * N_PAD + src, pad_iota % N_ENTITY]).reshape(-1, 128)
    dst_p = jnp.concatenate(
        [dst, N_ENTITY + pad_iota % (N_PAD - N_ENTITY)]).reshape(-1, 128)

    agg, deg = _edge_scatter_fn()(gidx_p, dst_p, z_flat)
    kg = _combine(agg, jnp.transpose(deg), node_emb_p, root_W,
                  root_b.reshape(1, D))

    uflat = user_ids.reshape(-1).astype(jnp.int32)
    upad = jnp.concatenate(
        [uflat,
         jnp.arange(U_PAD - BATCH * HIST, dtype=jnp.int32) % N_ENTITY])
    h_gathered = _user_gather_fn()(upad, kg)

    mask = jnp.asarray(_POOL_MASK)
    h = _attn_fc1(h_gathered, attn_Wa, attn_a.reshape(D, 1), mask,
                  fc1_W, fc1_b.reshape(1, D))
    return _bn_fc2(h, bn_gamma.reshape(1, D), bn_beta.reshape(1, D),
                   fc2_W, fc2_b.reshape(1, D))
